# baseline (speedup 1.0000x reference)
import functools

import jax
import jax.numpy as jnp
from jax import lax
from jax.experimental import pallas as pl
from jax.experimental.pallas import tpu as pltpu
from jax.experimental.pallas import tpu_sc as plsc

_N = 10000
_B = 8
_NPG = 1250
_D = 128
_KC = 16
_K = 4
_NC = 2
_NS = 16
_NW = _NC * _NS


def _sc_scatter_rows(values, src_idx, dst_idx, n_out, chunk, use_ones,
                     stage_table=False):
    e_tot = src_idx.shape[0]
    w = values.shape[1]
    per_tile = e_tot // _NW
    assert per_tile * _NW == e_tot and per_tile % chunk == 0
    nchunks = per_tile // chunk
    rows_out = -(-n_out // (_NS * chunk)) * chunk
    n_acc = rows_out * _NS
    n_full = rows_out // chunk
    n_vals = values.shape[0]
    if stage_table:
        assert n_vals % _NS == 0
    vrows_tile = n_vals // _NS
    src3 = src_idx.reshape(_NW, nchunks, chunk)
    dst3 = dst_idx.reshape(_NW, nchunks, chunk)
    fill = jnp.stack([jnp.zeros((chunk, w), jnp.float32),
                      jnp.ones((chunk, w), jnp.float32)])
    mesh = plsc.VectorSubcoreMesh(core_axis_name="c", subcore_axis_name="s")

    @functools.partial(
        pl.kernel,
        out_type=jax.ShapeDtypeStruct((_NC, n_acc, w), jnp.float32),
        mesh=mesh,
        compiler_params=pltpu.CompilerParams(use_tc_tiling_on_sc=False),
        scratch_types=[
            pltpu.VMEM((nchunks, chunk), jnp.int32),
            pltpu.VMEM((nchunks, chunk), jnp.int32),
            pltpu.VMEM((chunk, w), jnp.float32),
            pltpu.VMEM((chunk, w), jnp.float32),
            pltpu.VMEM_SHARED((n_acc, w), jnp.float32),
            pltpu.VMEM_SHARED((n_vals if stage_table else 8, w), jnp.float32),
            pltpu.SemaphoreType.DMA,
            pltpu.SemaphoreType.DMA,
        ],
    )
    def k(vals_hbm, src_hbm, dst_hbm, fill_hbm, out_hbm,
          src_v, dst_v, rows_a, rows_b, acc_sh, tbl_sh, sem_a, sem_b):
        vals = tbl_sh if stage_table else vals_hbm
        c = lax.axis_index("c")
        s = lax.axis_index("s")
        wid = c * _NS + s
        base_r = s * rows_out
        pltpu.sync_copy(fill_hbm.at[0], rows_a)

        def zero_body(j, carry):
            pltpu.sync_copy(rows_a, acc_sh.at[pl.ds(base_r + j * chunk, chunk)])
            return carry

        lax.fori_loop(0, n_full, zero_body, 0)
        if stage_table:
            pltpu.sync_copy(vals_hbm.at[pl.ds(s * vrows_tile, vrows_tile)],
                            tbl_sh.at[pl.ds(s * vrows_tile, vrows_tile)])
        pltpu.sync_copy(src_hbm.at[wid], src_v)
        pltpu.sync_copy(dst_hbm.at[wid], dst_v)
        plsc.subcore_barrier()

        if use_ones:
            pltpu.sync_copy(fill_hbm.at[1], rows_a)

            def chunk_body(j, carry):
                pltpu.sync_copy(rows_a, acc_sh.at[dst_v.at[j]], add=True)
                return carry

            lax.fori_loop(0, nchunks, chunk_body, 0)
        else:
            bufs = (rows_a, rows_b)
            sems = (sem_a, sem_b)
            pltpu.async_copy(vals.at[src_v.at[0]], rows_a, sem_a)

            def chunk_body(j, carry):
                for par in (0, 1):
                    @pl.when(j % 2 == par)
                    def _():
                        buf, sem = bufs[par], sems[par]
                        nbuf, nsem = bufs[1 - par], sems[1 - par]
                        pltpu.make_async_copy(
                            vals_hbm.at[pl.ds(0, chunk)], buf, sem).wait()

                        @pl.when(j + 1 < nchunks)
                        def _():
                            pltpu.async_copy(
                                vals.at[src_v.at[j + 1]], nbuf, nsem)
                        pltpu.sync_copy(buf, acc_sh.at[dst_v.at[j]], add=True)
                return carry

            lax.fori_loop(0, nchunks, chunk_body, 0)
        plsc.subcore_barrier()
        pltpu.sync_copy(acc_sh.at[pl.ds(base_r, rows_out)],
                        out_hbm.at[c, pl.ds(base_r, rows_out)])

    return k(values, src3, dst3, fill)


_ROWS = 1000


def _bdot(a, b, dims=(((1,), (0,)), ((), ()))):
    return lax.dot_general(a.astype(jnp.bfloat16), b.astype(jnp.bfloat16),
                           dims, preferred_element_type=jnp.float32)


def _mms_body(x_ref, w_ref, c_ref, o_ref):
    cnt = c_ref[0, :, 0:1] + c_ref[1, :, 0:1]
    dinv = lax.rsqrt(cnt + 1.0)
    o_ref[...] = _bdot(x_ref[...], w_ref[...]) * dinv


def _mm_scale(x, w, cntp):
    return pl.pallas_call(
        _mms_body,
        grid=(_N // _ROWS,),
        in_specs=[pl.BlockSpec((_ROWS, _D), lambda i: (i, 0)),
                  pl.BlockSpec((_D, _D), lambda i: (0, 0)),
                  pl.BlockSpec((2, _ROWS, _KC), lambda i: (0, i, 0))],
        out_specs=pl.BlockSpec((_ROWS, _D), lambda i: (i, 0)),
        out_shape=jax.ShapeDtypeStruct((_N, _D), jnp.float32),
    )(x, w, cntp)


def _conv1_body(hws_ref, agg_ref, c_ref, b1_ref, w2_ref, h_ref, hw2_ref):
    cnt = c_ref[0, :, 0:1] + c_ref[1, :, 0:1]
    dinv = lax.rsqrt(cnt + 1.0)
    tot = agg_ref[0] + agg_ref[1] + hws_ref[...]
    h = jnp.maximum(dinv * tot + b1_ref[...], 0.0)
    h_ref[...] = h
    hw2_ref[...] = _bdot(h, w2_ref[...])


def _conv1_finish(hws, aggp, cntp, b1, w2):
    return pl.pallas_call(
        _conv1_body,
        grid=(_N // _ROWS,),
        in_specs=[pl.BlockSpec((_ROWS, _D), lambda i: (i, 0)),
                  pl.BlockSpec((2, _ROWS, _D), lambda i: (0, i, 0)),
                  pl.BlockSpec((2, _ROWS, _KC), lambda i: (0, i, 0)),
                  pl.BlockSpec((1, _D), lambda i: (0, 0)),
                  pl.BlockSpec((_D, _D), lambda i: (0, 0))],
        out_specs=[pl.BlockSpec((_ROWS, _D), lambda i: (i, 0)),
                   pl.BlockSpec((_ROWS, _D), lambda i: (i, 0))],
        out_shape=[jax.ShapeDtypeStruct((_N, _D), jnp.float32),
                   jax.ShapeDtypeStruct((_N, _D), jnp.float32)],
    )(hws, aggp, cntp, b1.reshape(1, _D), w2)


def _sqrow_body(h_ref, o_ref):
    p = h_ref[0]
    col128 = lax.broadcasted_iota(jnp.int32, (_NPG, _D), 1)
    pm = jnp.where(col128 < 3, p, 0.0)
    sq_col = jnp.sum(pm * pm, axis=1, keepdims=True)
    rows_i = lax.broadcasted_iota(jnp.int32, (_NPG, _NPG), 0)
    cols_i = lax.broadcasted_iota(jnp.int32, (_NPG, _NPG), 1)
    eyef = jnp.where(rows_i == cols_i, 1.0, 0.0)
    o_ref[0] = lax.dot_general(sq_col, eyef, (((0,), (0,)), ((), ())),
                               precision=lax.Precision.HIGHEST,
                               preferred_element_type=jnp.float32)


def _sqrow(h3):
    return pl.pallas_call(
        _sqrow_body,
        grid=(_B,),
        in_specs=[pl.BlockSpec((1, _NPG, _D), lambda b: (b, 0, 0))],
        out_specs=pl.BlockSpec((1, 1, _NPG), lambda b: (b, 0, 0)),
        out_shape=jax.ShapeDtypeStruct((_B, 1, _NPG), jnp.float32),
    )(h3)


_RB = 8


def _knn_body(hb_ref, ha_ref, sqr_ref, o_ref):
    hb = hb_ref[0]
    col128 = lax.broadcasted_iota(jnp.int32, (_RB, _D), 1)
    pmb = jnp.where(col128 < 3, hb, 0.0)
    g = _bdot(pmb, ha_ref[0], (((1,), (1,)), ((), ())))
    sqb = jnp.sum(pmb * pmb, axis=1, keepdims=True)
    d = sqb + sqr_ref[0] - 2.0 * g
    rows_g = (lax.broadcasted_iota(jnp.int32, (_RB, _NPG), 0)
              + pl.program_id(1) * _RB)
    cols_i = lax.broadcasted_iota(jnp.int32, (_RB, _NPG), 1)
    d = jnp.where(cols_i == rows_g, jnp.inf, d)
    ams = []
    for _ in range(_K):
        m = jnp.min(d, axis=1, keepdims=True)
        am = jnp.min(jnp.where(d == m, cols_i, jnp.int32(1 << 30)),
                     axis=1, keepdims=True)
        ams.append(am)
        d = jnp.where(cols_i == am, jnp.inf, d)
    nbr = jnp.concatenate(ams + ams[:_K], axis=1)
    o_ref[0] = nbr + pl.program_id(0) * _NPG


def _knn(h3, sqr):
    nrb = -(-_NPG // _RB)
    return pl.pallas_call(
        _knn_body,
        grid=(_B, nrb),
        in_specs=[pl.BlockSpec((1, _RB, _D), lambda b, r: (b, r, 0)),
                  pl.BlockSpec((1, _NPG, _D), lambda b, r: (b, 0, 0)),
                  pl.BlockSpec((1, 1, _NPG), lambda b, r: (b, 0, 0))],
        out_specs=pl.BlockSpec((1, _RB, 2 * _K), lambda b, r: (b, r, 0)),
        out_shape=jax.ShapeDtypeStruct((_B, _NPG, 2 * _K), jnp.int32),
    )(h3, h3, sqr)


def _conv2_body(hw2_ref, agg_ref, b2_ref, wp_ref, bp_ref, h2_ref, s_ref):
    tot = (hw2_ref[...] + agg_ref[0] + agg_ref[1]) * 0.2
    h2 = jnp.maximum(tot + b2_ref[...], 0.0)
    h2_ref[...] = h2
    logits = _bdot(h2, wp_ref[...]) + bp_ref[...]
    mx = jnp.max(logits, axis=1, keepdims=True)
    ex = jnp.exp(logits - mx)
    s_ref[...] = ex / jnp.sum(ex, axis=1, keepdims=True)


def _conv2_s(hw2, agg2p, b2, wp, bp):
    return pl.pallas_call(
        _conv2_body,
        grid=(_N // _ROWS,),
        in_specs=[pl.BlockSpec((_ROWS, _D), lambda i: (i, 0)),
                  pl.BlockSpec((2, _ROWS, _D), lambda i: (0, i, 0)),
                  pl.BlockSpec((1, _D), lambda i: (0, 0)),
                  pl.BlockSpec((_D, _KC), lambda i: (0, 0)),
                  pl.BlockSpec((1, _KC), lambda i: (0, 0))],
        out_specs=[pl.BlockSpec((_ROWS, _D), lambda i: (i, 0)),
                   pl.BlockSpec((_ROWS, _KC), lambda i: (i, 0))],
        out_shape=[jax.ShapeDtypeStruct((_N, _D), jnp.float32),
                   jax.ShapeDtypeStruct((_N, _KC), jnp.float32)],
    )(hw2, agg2p, b2.reshape(1, _D), wp, bp.reshape(1, _KC))


_SELU_ALPHA = 1.6732632423543772
_SELU_SCALE = 1.0507009873554805


def _pool_body(s_ref, h_ref, u_ref, c_ref, lsm_ref, scal_ref):
    sb = s_ref[0]
    h2b = h_ref[0]
    ub = u_ref[0, 0] + u_ref[1, 0]
    cntb = c_ref[0, 0, :, 0:1] + c_ref[1, 0, :, 0:1]
    dn = (((0,), (0,)), ((), ()))
    outb = _bdot(sb, h2b, dn)
    oadj = lax.dot_general(ub, sb, dn, preferred_element_type=jnp.float32)
    m = jnp.sum(cntb) * 0.5
    ca = lax.dot_general(sb, cntb, dn, preferred_element_type=jnp.float32)
    norm = lax.dot_general(ca, ca, (((1,), (1,)), ((), ())),
                           preferred_element_type=jnp.float32) / (2.0 * m)
    dec = oadj - norm
    r16 = lax.broadcasted_iota(jnp.int32, (_KC, _KC), 0)
    c16 = lax.broadcasted_iota(jnp.int32, (_KC, _KC), 1)
    eye16 = r16 == c16
    spec = -jnp.sum(jnp.where(eye16, dec, 0.0)) / (2.0 * m)
    ss = lax.dot_general(sb, sb, dn, preferred_element_type=jnp.float32)
    ssf = jnp.sqrt(jnp.sum(ss * ss))
    dif = ss / ssf - jnp.where(eye16, 0.25, 0.0)
    orth = jnp.sqrt(jnp.sum(dif * dif))
    csize = jnp.sum(sb, axis=0, keepdims=True)
    cl = jnp.sqrt(jnp.sum(csize * csize)) / _NPG * 4.0 - 1.0
    x = outb
    selu = _SELU_SCALE * jnp.where(x > 0.0, x,
                                   _SELU_ALPHA * (jnp.exp(x) - 1.0))
    mx = jnp.max(selu, axis=1, keepdims=True)
    sh = selu - mx
    lsm_ref[0] = sh - jnp.log(jnp.sum(jnp.exp(sh), axis=1, keepdims=True))
    lane = lax.broadcasted_iota(jnp.int32, (1, 1, _D), 2)
    scal_ref[...] = jnp.where(lane == 0, spec,
                              jnp.where(lane == 1, orth,
                                        jnp.where(lane == 2, cl, 0.0)))


def _pool(s3, h23, u4, c4):
    return pl.pallas_call(
        _pool_body,
        grid=(_B,),
        in_specs=[pl.BlockSpec((1, _NPG, _KC), lambda b: (b, 0, 0)),
                  pl.BlockSpec((1, _NPG, _D), lambda b: (b, 0, 0)),
                  pl.BlockSpec((2, 1, _NPG, _KC), lambda b: (0, b, 0, 0)),
                  pl.BlockSpec((2, 1, _NPG, _KC), lambda b: (0, b, 0, 0))],
        out_specs=[pl.BlockSpec((1, _KC, _D), lambda b: (b, 0, 0)),
                   pl.BlockSpec((1, 1, _D), lambda b: (b, 0, 0))],
        out_shape=[jax.ShapeDtypeStruct((_B, _KC, _D), jnp.float32),
                   jax.ShapeDtypeStruct((_B, 1, _D), jnp.float32)],
    )(s3, h23, u4, c4)


def kernel(x, edge_index, batch, W1, b1, W2, b2, Wp, bp):
    src = edge_index[0]
    dst = edge_index[1]
    ones_tbl = jnp.ones((8, _KC), jnp.float32)
    zsrc = jnp.zeros_like(src)
    cntp = _sc_scatter_rows(ones_tbl, zsrc, dst, _N, 80, use_ones=True)
    hws = _mm_scale(x, W1, cntp)
    aggp = _sc_scatter_rows(hws, src, dst, _N, 80, use_ones=False)
    h, hw2 = _conv1_finish(hws, aggp, cntp, b1, W2)
    h3 = h.reshape(_B, _NPG, _D)
    nbr8 = _knn(h3, _sqrow(h3))
    src2 = nbr8[..., :_K].reshape(-1)
    pad = 32 * 1280 - _N * _K
    src2p = jnp.concatenate([src2, jnp.zeros((pad,), jnp.int32)])
    dst2 = jnp.repeat(jnp.arange(_N, dtype=jnp.int32), _K)
    dst2p = jnp.concatenate([dst2, jnp.full((pad,), _N, jnp.int32)])
    agg2p = _sc_scatter_rows(hw2, src2p, dst2p, _N, 80, use_ones=False)
    h2, s = _conv2_s(hw2, agg2p, b2, Wp, bp)
    up = _sc_scatter_rows(s, src, dst, _N, 80, use_ones=False,
                          stage_table=True)
    lsm, scal = _pool(s.reshape(_B, _NPG, _KC),
                      h2.reshape(_B, _NPG, _D),
                      up[:, :_N].reshape(2, _B, _NPG, _KC),
                      cntp[:, :_N].reshape(2, _B, _NPG, _KC))
    loss = scal[:, 0, 0].mean() + scal[:, 0, 1].mean() + scal[:, 0, 2].mean()
    return lsm, loss, s.reshape(_B, _NPG, _KC)


# speedup vs baseline: 3.8362x; 3.8362x over previous
import functools

import jax
import jax.numpy as jnp
from jax import lax
from jax.experimental import pallas as pl
from jax.experimental.pallas import tpu as pltpu
from jax.experimental.pallas import tpu_sc as plsc

_N = 10000
_B = 8
_NPG = 1250
_D = 128
_KC = 16
_K = 4
_NC = 2
_NS = 16
_NW = _NC * _NS


def _sc_scatter_rows(values, src_idx, dst_idx, n_out, chunk, use_ones,
                     stage_table=False):
    e_tot = src_idx.shape[0]
    w = values.shape[1]
    per_tile = e_tot // _NW
    assert per_tile * _NW == e_tot and per_tile % chunk == 0
    nchunks = per_tile // chunk
    rows_out = -(-n_out // (_NS * chunk)) * chunk
    n_acc = rows_out * _NS
    n_full = rows_out // chunk
    n_vals = values.shape[0]
    if stage_table:
        assert n_vals % _NS == 0
    vrows_tile = n_vals // _NS
    src3 = src_idx.reshape(_NW, nchunks, chunk)
    dst3 = dst_idx.reshape(_NW, nchunks, chunk)
    fill = jnp.stack([jnp.zeros((chunk, w), jnp.float32),
                      jnp.ones((chunk, w), jnp.float32)])
    mesh = plsc.VectorSubcoreMesh(core_axis_name="c", subcore_axis_name="s")

    @functools.partial(
        pl.kernel,
        out_type=jax.ShapeDtypeStruct((_NC, n_acc, w), jnp.float32),
        mesh=mesh,
        compiler_params=pltpu.CompilerParams(use_tc_tiling_on_sc=False),
        scratch_types=[
            pltpu.VMEM((nchunks, chunk), jnp.int32),
            pltpu.VMEM((nchunks, chunk), jnp.int32),
            pltpu.VMEM((chunk, w), jnp.float32),
            pltpu.VMEM((chunk, w), jnp.float32),
            pltpu.VMEM_SHARED((n_acc, w), jnp.float32),
            pltpu.VMEM_SHARED((n_vals if stage_table else 8, w), jnp.float32),
            pltpu.SemaphoreType.DMA,
            pltpu.SemaphoreType.DMA,
        ],
    )
    def k(vals_hbm, src_hbm, dst_hbm, fill_hbm, out_hbm,
          src_v, dst_v, rows_a, rows_b, acc_sh, tbl_sh, sem_a, sem_b):
        vals = tbl_sh if stage_table else vals_hbm
        c = lax.axis_index("c")
        s = lax.axis_index("s")
        wid = c * _NS + s
        base_r = s * rows_out
        pltpu.sync_copy(fill_hbm.at[0], rows_a)

        def zero_body(j, carry):
            pltpu.sync_copy(rows_a, acc_sh.at[pl.ds(base_r + j * chunk, chunk)])
            return carry

        lax.fori_loop(0, n_full, zero_body, 0)
        if stage_table:
            pltpu.sync_copy(vals_hbm.at[pl.ds(s * vrows_tile, vrows_tile)],
                            tbl_sh.at[pl.ds(s * vrows_tile, vrows_tile)])
        pltpu.sync_copy(src_hbm.at[wid], src_v)
        pltpu.sync_copy(dst_hbm.at[wid], dst_v)
        plsc.subcore_barrier()

        if use_ones:
            pltpu.sync_copy(fill_hbm.at[1], rows_a)

            def chunk_body(j, carry):
                pltpu.sync_copy(rows_a, acc_sh.at[dst_v.at[j]], add=True)
                return carry

            lax.fori_loop(0, nchunks, chunk_body, 0)
        else:
            bufs = (rows_a, rows_b)
            sems = (sem_a, sem_b)
            pltpu.async_copy(vals.at[src_v.at[0]], rows_a, sem_a)

            def chunk_body(j, carry):
                for par in (0, 1):
                    @pl.when(j % 2 == par)
                    def _():
                        buf, sem = bufs[par], sems[par]
                        nbuf, nsem = bufs[1 - par], sems[1 - par]
                        pltpu.make_async_copy(
                            vals_hbm.at[pl.ds(0, chunk)], buf, sem).wait()

                        @pl.when(j + 1 < nchunks)
                        def _():
                            pltpu.async_copy(
                                vals.at[src_v.at[j + 1]], nbuf, nsem)
                        pltpu.sync_copy(buf, acc_sh.at[dst_v.at[j]], add=True)
                return carry

            lax.fori_loop(0, nchunks, chunk_body, 0)
        plsc.subcore_barrier()
        pltpu.sync_copy(acc_sh.at[pl.ds(base_r, rows_out)],
                        out_hbm.at[c, pl.ds(base_r, rows_out)])

    return k(values, src3, dst3, fill)


_ROWS = 1000


def _bdot(a, b, dims=(((1,), (0,)), ((), ()))):
    return lax.dot_general(a.astype(jnp.bfloat16), b.astype(jnp.bfloat16),
                           dims, preferred_element_type=jnp.float32)


def _mms_body(x_ref, w_ref, c_ref, o_ref):
    cnt = c_ref[0, :, 0:1] + c_ref[1, :, 0:1]
    dinv = lax.rsqrt(cnt + 1.0)
    o_ref[...] = _bdot(x_ref[...], w_ref[...]) * dinv


def _mm_scale(x, w, cntp):
    return pl.pallas_call(
        _mms_body,
        grid=(_N // _ROWS,),
        in_specs=[pl.BlockSpec((_ROWS, _D), lambda i: (i, 0)),
                  pl.BlockSpec((_D, _D), lambda i: (0, 0)),
                  pl.BlockSpec((2, _ROWS, _KC), lambda i: (0, i, 0))],
        out_specs=pl.BlockSpec((_ROWS, _D), lambda i: (i, 0)),
        out_shape=jax.ShapeDtypeStruct((_N, _D), jnp.float32),
    )(x, w, cntp)


def _conv1_body(hws_ref, agg_ref, c_ref, b1_ref, w2_ref, h_ref, hw2_ref):
    cnt = c_ref[0, :, 0:1] + c_ref[1, :, 0:1]
    dinv = lax.rsqrt(cnt + 1.0)
    tot = agg_ref[0] + agg_ref[1] + hws_ref[...]
    h = jnp.maximum(dinv * tot + b1_ref[...], 0.0)
    h_ref[...] = h
    hw2_ref[...] = _bdot(h, w2_ref[...])


def _conv1_finish(hws, aggp, cntp, b1, w2):
    return pl.pallas_call(
        _conv1_body,
        grid=(_N // _ROWS,),
        in_specs=[pl.BlockSpec((_ROWS, _D), lambda i: (i, 0)),
                  pl.BlockSpec((2, _ROWS, _D), lambda i: (0, i, 0)),
                  pl.BlockSpec((2, _ROWS, _KC), lambda i: (0, i, 0)),
                  pl.BlockSpec((1, _D), lambda i: (0, 0)),
                  pl.BlockSpec((_D, _D), lambda i: (0, 0))],
        out_specs=[pl.BlockSpec((_ROWS, _D), lambda i: (i, 0)),
                   pl.BlockSpec((_ROWS, _D), lambda i: (i, 0))],
        out_shape=[jax.ShapeDtypeStruct((_N, _D), jnp.float32),
                   jax.ShapeDtypeStruct((_N, _D), jnp.float32)],
    )(hws, aggp, cntp, b1.reshape(1, _D), w2)


def _sqrow_body(h_ref, o_ref):
    p = h_ref[0]
    col128 = lax.broadcasted_iota(jnp.int32, (_NPG, _D), 1)
    pm = jnp.where(col128 < 3, p, 0.0)
    sq_col = jnp.sum(pm * pm, axis=1, keepdims=True)
    rows_i = lax.broadcasted_iota(jnp.int32, (_NPG, _NPG), 0)
    cols_i = lax.broadcasted_iota(jnp.int32, (_NPG, _NPG), 1)
    eyef = jnp.where(rows_i == cols_i, 1.0, 0.0)
    o_ref[0] = lax.dot_general(sq_col, eyef, (((0,), (0,)), ((), ())),
                               precision=lax.Precision.HIGHEST,
                               preferred_element_type=jnp.float32)


def _sqrow(h3):
    return pl.pallas_call(
        _sqrow_body,
        grid=(_B,),
        in_specs=[pl.BlockSpec((1, _NPG, _D), lambda b: (b, 0, 0))],
        out_specs=pl.BlockSpec((1, 1, _NPG), lambda b: (b, 0, 0)),
        out_shape=jax.ShapeDtypeStruct((_B, 1, _NPG), jnp.float32),
    )(h3)


def _knn_body(hb_ref, sqr_ref, o_ref):
    hb = hb_ref[0]
    col128 = lax.broadcasted_iota(jnp.int32, (_NPG, _D), 1)
    pmb = jnp.where(col128 < 3, hb, 0.0)
    g = _bdot(pmb, hb, (((1,), (1,)), ((), ())))
    sqb = jnp.sum(pmb * pmb, axis=1, keepdims=True)
    d = sqb + sqr_ref[0] - 2.0 * g
    rows_g = lax.broadcasted_iota(jnp.int32, (_NPG, _NPG), 0)
    cols_i = lax.broadcasted_iota(jnp.int32, (_NPG, _NPG), 1)
    d = jnp.where(cols_i == rows_g, jnp.inf, d)
    ams = []
    for _ in range(_K):
        m = jnp.min(d, axis=1, keepdims=True)
        am = jnp.min(jnp.where(d == m, cols_i, jnp.int32(1 << 30)),
                     axis=1, keepdims=True)
        ams.append(am)
        d = jnp.where(cols_i == am, jnp.inf, d)
    nbr = jnp.concatenate(ams + ams[:_K], axis=1)
    o_ref[0] = nbr + pl.program_id(0) * _NPG


def _knn(h3, sqr):
    return pl.pallas_call(
        _knn_body,
        grid=(_B,),
        in_specs=[pl.BlockSpec((1, _NPG, _D), lambda b: (b, 0, 0)),
                  pl.BlockSpec((1, 1, _NPG), lambda b: (b, 0, 0))],
        out_specs=pl.BlockSpec((1, _NPG, 2 * _K), lambda b: (b, 0, 0)),
        out_shape=jax.ShapeDtypeStruct((_B, _NPG, 2 * _K), jnp.int32),
    )(h3, sqr)


def _conv2_body(hw2_ref, agg_ref, b2_ref, wp_ref, bp_ref, h2_ref, s_ref):
    tot = (hw2_ref[...] + agg_ref[0] + agg_ref[1]) * 0.2
    h2 = jnp.maximum(tot + b2_ref[...], 0.0)
    h2_ref[...] = h2
    logits = _bdot(h2, wp_ref[...]) + bp_ref[...]
    mx = jnp.max(logits, axis=1, keepdims=True)
    ex = jnp.exp(logits - mx)
    s_ref[...] = ex / jnp.sum(ex, axis=1, keepdims=True)


def _conv2_s(hw2, agg2p, b2, wp, bp):
    return pl.pallas_call(
        _conv2_body,
        grid=(_N // _ROWS,),
        in_specs=[pl.BlockSpec((_ROWS, _D), lambda i: (i, 0)),
                  pl.BlockSpec((2, _ROWS, _D), lambda i: (0, i, 0)),
                  pl.BlockSpec((1, _D), lambda i: (0, 0)),
                  pl.BlockSpec((_D, _KC), lambda i: (0, 0)),
                  pl.BlockSpec((1, _KC), lambda i: (0, 0))],
        out_specs=[pl.BlockSpec((_ROWS, _D), lambda i: (i, 0)),
                   pl.BlockSpec((_ROWS, _KC), lambda i: (i, 0))],
        out_shape=[jax.ShapeDtypeStruct((_N, _D), jnp.float32),
                   jax.ShapeDtypeStruct((_N, _KC), jnp.float32)],
    )(hw2, agg2p, b2.reshape(1, _D), wp, bp.reshape(1, _KC))


_SELU_ALPHA = 1.6732632423543772
_SELU_SCALE = 1.0507009873554805


def _pool_body(s_ref, h_ref, u_ref, c_ref, lsm_ref, scal_ref):
    sb = s_ref[0]
    h2b = h_ref[0]
    ub = u_ref[0, 0] + u_ref[1, 0]
    cntb = c_ref[0, 0, :, 0:1] + c_ref[1, 0, :, 0:1]
    dn = (((0,), (0,)), ((), ()))
    outb = _bdot(sb, h2b, dn)
    oadj = lax.dot_general(ub, sb, dn, preferred_element_type=jnp.float32)
    m = jnp.sum(cntb) * 0.5
    ca = lax.dot_general(sb, cntb, dn, preferred_element_type=jnp.float32)
    norm = lax.dot_general(ca, ca, (((1,), (1,)), ((), ())),
                           preferred_element_type=jnp.float32) / (2.0 * m)
    dec = oadj - norm
    r16 = lax.broadcasted_iota(jnp.int32, (_KC, _KC), 0)
    c16 = lax.broadcasted_iota(jnp.int32, (_KC, _KC), 1)
    eye16 = r16 == c16
    spec = -jnp.sum(jnp.where(eye16, dec, 0.0)) / (2.0 * m)
    ss = lax.dot_general(sb, sb, dn, preferred_element_type=jnp.float32)
    ssf = jnp.sqrt(jnp.sum(ss * ss))
    dif = ss / ssf - jnp.where(eye16, 0.25, 0.0)
    orth = jnp.sqrt(jnp.sum(dif * dif))
    csize = jnp.sum(sb, axis=0, keepdims=True)
    cl = jnp.sqrt(jnp.sum(csize * csize)) / _NPG * 4.0 - 1.0
    x = outb
    selu = _SELU_SCALE * jnp.where(x > 0.0, x,
                                   _SELU_ALPHA * (jnp.exp(x) - 1.0))
    mx = jnp.max(selu, axis=1, keepdims=True)
    sh = selu - mx
    lsm_ref[0] = sh - jnp.log(jnp.sum(jnp.exp(sh), axis=1, keepdims=True))
    lane = lax.broadcasted_iota(jnp.int32, (1, 1, _D), 2)
    scal_ref[...] = jnp.where(lane == 0, spec,
                              jnp.where(lane == 1, orth,
                                        jnp.where(lane == 2, cl, 0.0)))


def _pool(s3, h23, u4, c4):
    return pl.pallas_call(
        _pool_body,
        grid=(_B,),
        in_specs=[pl.BlockSpec((1, _NPG, _KC), lambda b: (b, 0, 0)),
                  pl.BlockSpec((1, _NPG, _D), lambda b: (b, 0, 0)),
                  pl.BlockSpec((2, 1, _NPG, _KC), lambda b: (0, b, 0, 0)),
                  pl.BlockSpec((2, 1, _NPG, _KC), lambda b: (0, b, 0, 0))],
        out_specs=[pl.BlockSpec((1, _KC, _D), lambda b: (b, 0, 0)),
                   pl.BlockSpec((1, 1, _D), lambda b: (b, 0, 0))],
        out_shape=[jax.ShapeDtypeStruct((_B, _KC, _D), jnp.float32),
                   jax.ShapeDtypeStruct((_B, 1, _D), jnp.float32)],
    )(s3, h23, u4, c4)


def kernel(x, edge_index, batch, W1, b1, W2, b2, Wp, bp):
    src = edge_index[0]
    dst = edge_index[1]
    ones_tbl = jnp.ones((8, _KC), jnp.float32)
    zsrc = jnp.zeros_like(src)
    cntp = _sc_scatter_rows(ones_tbl, zsrc, dst, _N, 80, use_ones=True)
    hws = _mm_scale(x, W1, cntp)
    aggp = _sc_scatter_rows(hws, src, dst, _N, 80, use_ones=False)
    h, hw2 = _conv1_finish(hws, aggp, cntp, b1, W2)
    h3 = h.reshape(_B, _NPG, _D)
    nbr8 = _knn(h3, _sqrow(h3))
    src2 = nbr8[..., :_K].reshape(-1)
    pad = 32 * 1280 - _N * _K
    src2p = jnp.concatenate([src2, jnp.zeros((pad,), jnp.int32)])
    dst2 = jnp.repeat(jnp.arange(_N, dtype=jnp.int32), _K)
    dst2p = jnp.concatenate([dst2, jnp.full((pad,), _N, jnp.int32)])
    agg2p = _sc_scatter_rows(hw2, src2p, dst2p, _N, 80, use_ones=False)
    h2, s = _conv2_s(hw2, agg2p, b2, Wp, bp)
    up = _sc_scatter_rows(s, src, dst, _N, 80, use_ones=False,
                          stage_table=True)
    lsm, scal = _pool(s.reshape(_B, _NPG, _KC),
                      h2.reshape(_B, _NPG, _D),
                      up[:, :_N].reshape(2, _B, _NPG, _KC),
                      cntp[:, :_N].reshape(2, _B, _NPG, _KC))
    loss = scal[:, 0, 0].mean() + scal[:, 0, 1].mean() + scal[:, 0, 2].mean()
    return lsm, loss, s.reshape(_B, _NPG, _KC)
